# baseline (speedup 1.0000x reference)
import functools

import jax
import jax.numpy as jnp
from jax import lax
from jax.experimental import pallas as pl
from jax.experimental.pallas import tpu as pltpu
from jax.experimental.pallas import tpu_sc as plsc

_NUM_CORES = 2
_NUM_SUBCORES = 16
_NW = _NUM_CORES * _NUM_SUBCORES
_CHUNK = 128
_L = 16


@functools.lru_cache(maxsize=None)
def _build(B, NQ2, NT8):
    b_per_w = B // _NW
    n_grp = b_per_w // _L
    n_chunk = b_per_w // _CHUNK
    mesh = plsc.VectorSubcoreMesh(core_axis_name="c", subcore_axis_name="s")

    @functools.partial(
        pl.kernel,
        mesh=mesh,
        compiler_params=pltpu.CompilerParams(
            use_tc_tiling_on_sc=False, needs_layout_passes=False),
        out_type=(
            jax.ShapeDtypeStruct((B, 4), jnp.float32),
            jax.ShapeDtypeStruct((B, 3), jnp.float32),
        ),
        scratch_types=[
            pltpu.VMEM((b_per_w,), jnp.int32),
            pltpu.VMEM((n_chunk, _CHUNK), jnp.int32),
            pltpu.VMEM((n_chunk, _CHUNK), jnp.int32),
            pltpu.VMEM((n_chunk, _CHUNK), jnp.int32),
            pltpu.VMEM((b_per_w, 8), jnp.float32),
            pltpu.VMEM((b_per_w, 8), jnp.float32),
            pltpu.VMEM((b_per_w, 8), jnp.float32),
            pltpu.VMEM((b_per_w, 4), jnp.float32),
            pltpu.VMEM((b_per_w, 3), jnp.float32),
            pltpu.SemaphoreType.DMA,
        ],
    )
    def k(idx_hbm, qv_hbm, tv_hbm, q_out, t_out,
          idx_v, pq_v, pa_v, pb_v, bq, ba, bb, oq, ot, sem):
        wid = lax.axis_index("s") * _NUM_CORES + lax.axis_index("c")
        base = wid * b_per_w
        pltpu.sync_copy(idx_hbm.at[pl.ds(base, b_per_w)], idx_v)

        for g in range(n_grp):
            sl = pl.ds(g * _L, _L)
            c, col = g // (_CHUNK // _L), (g % (_CHUNK // _L)) * _L
            r = idx_v[sl]
            pq_v[c, pl.ds(col, _L)] = lax.shift_right_logical(r, 1)
            pa = lax.shift_right_logical(r * 3, 3)
            pa_v[c, pl.ds(col, _L)] = pa
            pb_v[c, pl.ds(col, _L)] = jnp.minimum(pa + 1, NT8 - 1)

        copies = []
        for c in range(n_chunk):
            dst = pl.ds(c * _CHUNK, _CHUNK)
            copies.append(
                pltpu.async_copy(qv_hbm.at[pq_v.at[c]], bq.at[dst], sem))
            copies.append(
                pltpu.async_copy(tv_hbm.at[pa_v.at[c]], ba.at[dst], sem))
            copies.append(
                pltpu.async_copy(tv_hbm.at[pb_v.at[c]], bb.at[dst], sem))
        for cp in copies:
            cp.wait()

        iota = lax.iota(jnp.int32, _L)
        for g in range(n_grp):
            sl = pl.ds(g * _L, _L)
            kvec = g * _L + iota
            r = idx_v[sl]
            qcol = (r & 1) * 4
            for j in range(4):
                vals = plsc.load_gather(bq, [kvec, qcol + j])
                plsc.store_scatter(oq, [kvec, iota * 0 + j], vals)
            w3 = r * 3
            la = w3 - lax.shift_left(lax.shift_right_logical(w3, 3), 3)
            for j in range(3):
                loc = la + j
                va = plsc.load_gather(ba, [kvec, jnp.minimum(loc, 7)])
                vb = plsc.load_gather(bb, [kvec, jnp.maximum(loc - 8, 0)])
                vals = jnp.where(loc < 8, va, vb)
                plsc.store_scatter(ot, [kvec, iota * 0 + j], vals)

        pltpu.sync_copy(oq, q_out.at[pl.ds(base, b_per_w)])
        pltpu.sync_copy(ot, t_out.at[pl.ds(base, b_per_w)])

    return k


def kernel(indices, q, t):
    B = indices.shape[0]
    N = q.shape[0]
    qv = q.reshape(N // 2, 8)
    tv = t.reshape(N * 3 // 8, 8)
    k = _build(B, N // 2, N * 3 // 8)
    q_sel, t_sel = k(indices.astype(jnp.int32), qv, tv)
    return (q_sel, t_sel)
